# baseline (speedup 1.0000x reference)
import jax
import jax.numpy as jnp
from jax.experimental import pallas as pl
from jax.experimental.pallas import tpu as pltpu

_N = 1_000_000
_F = 250_000
_BLK = 122_880
_GRID = -(-_N // _BLK)
_FBLK = 30_720


def _copy_body(*refs):
    n = len(refs) // 2
    for i in range(n):
        refs[n + i][...] = refs[i][...]


def kernel(new_xyz, new_colors, new_rots, new_scales, new_z_values,
           new_trackable_filter, new_voxel_index):
    args = (new_xyz.T, new_colors.T, new_rots.T, new_scales.T,
            new_z_values, new_trackable_filter, new_voxel_index)

    def _spec(shape):
        if len(shape) == 2:
            return pl.BlockSpec((shape[0], _BLK), lambda i: (0, i))
        if shape[0] == _F:
            return pl.BlockSpec((_FBLK,), lambda i: (i,))
        return pl.BlockSpec((_BLK,), lambda i: (i,))

    specs = [_spec(a.shape) for a in args]
    out_shape = tuple(jax.ShapeDtypeStruct(a.shape, a.dtype) for a in args)
    outs = pl.pallas_call(
        _copy_body,
        grid=(_GRID,),
        out_shape=out_shape,
        in_specs=specs,
        out_specs=specs,
    )(*args)
    return (outs[0].T, outs[1].T, outs[2].T, outs[3].T, outs[4], outs[5],
            outs[6])


# speedup vs baseline: 1.0133x; 1.0133x over previous
import jax
import jax.numpy as jnp
from jax.experimental import pallas as pl
from jax.experimental.pallas import tpu as pltpu

_N = 1_000_000
_F = 250_000
_BLK = 118_784
_GRID = -(-_N // _BLK)
_FBLK = 29_696


def _copy_body(*refs):
    n = len(refs) // 2
    for i in range(n):
        refs[n + i][...] = refs[i][...]


def kernel(new_xyz, new_colors, new_rots, new_scales, new_z_values,
           new_trackable_filter, new_voxel_index):
    args = (new_xyz.T, new_colors.T, new_rots.T, new_scales.T,
            new_z_values, new_trackable_filter, new_voxel_index)

    def _spec(shape):
        if len(shape) == 2:
            return pl.BlockSpec((shape[0], _BLK), lambda i: (0, i))
        if shape[0] == _F:
            return pl.BlockSpec((_FBLK,), lambda i: (i,))
        return pl.BlockSpec((_BLK,), lambda i: (i,))

    specs = [_spec(a.shape) for a in args]
    out_shape = tuple(jax.ShapeDtypeStruct(a.shape, a.dtype) for a in args)
    outs = pl.pallas_call(
        _copy_body,
        grid=(_GRID,),
        out_shape=out_shape,
        in_specs=specs,
        out_specs=specs,
    )(*args)
    return (outs[0].T, outs[1].T, outs[2].T, outs[3].T, outs[4], outs[5],
            outs[6])
